# baseline (speedup 1.0000x reference)
import jax
import jax.numpy as jnp
from jax import lax
from jax.experimental import pallas as pl
from jax.experimental.pallas import tpu as pltpu
from jax.experimental.pallas import tpu_sc as plsc

B = 16384
S = 4096
CB = 2048
NC = 2
NS = 16
NW = NC * NS
RPW = S // NW
G = RPW // 16


def _sc_body(pbT, psT, pbmT, psmT, tbl_hbm, pb_out, ps_out,
             pbv, psv, pbmv, psmv, tblv, sem):
    wid = lax.axis_index("s") * NC + lax.axis_index("c")
    base = wid * RPW

    cps = [
        pltpu.async_copy(pbT.at[:, pl.ds(base, RPW)], pbv, sem),
        pltpu.async_copy(pbmT.at[:, pl.ds(base, RPW)], pbmv, sem),
        pltpu.async_copy(psT.at[:, pl.ds(base, RPW)], psv, sem),
        pltpu.async_copy(psmT.at[:, pl.ds(base, RPW)], psmv, sem),
        pltpu.async_copy(tbl_hbm, tblv, sem),
    ]
    for c in cps:
        c.wait()

    def const(k):
        return tblv[k, :]

    d_pb = [const(k) for k in range(9)]
    d_def = [const(9 + k) for k in range(2)]
    d_nw = [const(11 + k) for k in range(2)]
    d_w = [const(13 + k) for k in range(2)]
    d_h1tt = [const(15 + k) for k in range(2)]
    d_h1tt_off = [const(17 + k) for k in range(2)]
    d_h1c = [const(19 + k) for k in range(3)]
    d_h1c_on = [const(22 + k) for k in range(3)]
    d_h1c_off = [const(25 + k) for k in range(3)]
    d_h2tt = [const(28 + k) for k in range(2)]
    d_h2tt_off = [const(30 + k) for k in range(2)]
    d_h2c = [const(32 + k) for k in range(2)]
    d_h2c_on = [const(34 + k) for k in range(2)]
    d_h2c_off = [const(36 + k) for k in range(2)]
    d_ps = [const(38 + k) for k in range(6)]

    def group(g, carry):
        off = g * 16

        def ld(buf, j):
            return buf[j, pl.ds(off, 16)]

        mb = [ld(pbmv, j) for j in range(9)]
        b0, b1, b2, b6 = ld(pbv, 0), ld(pbv, 1), ld(pbv, 2), ld(pbv, 6)
        pb1_0 = b0 * mb[0] + (1.0 - mb[0]) * d_pb[0]
        pb1_1 = b1 * mb[1] + (1.0 - mb[1]) * d_pb[1]
        pb1_2 = b2 * mb[2] + (1.0 - mb[2]) * d_pb[2]
        pb1_6 = b6 * mb[6] + (1.0 - mb[6]) * d_pb[6]

        has_nw = mb[0] > 0.5
        hot_nw = pb1_0 > 0.5
        has_w = mb[1] > 0.5
        hot_w = pb1_1 > 0.5
        ht1_known = mb[2] > 0.5
        ht1_hot = pb1_2 > 0.5
        ht2_known = mb[6] > 0.5
        ht2_hot = pb1_6 > 0.5

        pbv[0, pl.ds(off, 16)] = pb1_0
        pbv[1, pl.ds(off, 16)] = pb1_1
        pbv[2, pl.ds(off, 16)] = pb1_2
        pbv[6, pl.ds(off, 16)] = pb1_6
        for i, j in enumerate((3, 4, 5)):
            m = mb[j]
            pb1 = ld(pbv, j) * m + (1.0 - m) * d_pb[j]
            f = jnp.where(
                ht1_known,
                jnp.where(ht1_hot, d_h1c_on[i], d_h1c_off[i]), d_h1c[i])
            pbv[j, pl.ds(off, 16)] = pb1 * m + (1.0 - m) * f
        for i, j in enumerate((7, 8)):
            m = mb[j]
            pb1 = ld(pbv, j) * m + (1.0 - m) * d_pb[j]
            f = jnp.where(
                ht2_known,
                jnp.where(ht2_hot, d_h2c_on[i], d_h2c_off[i]), d_h2c[i])
            pbv[j, pl.ds(off, 16)] = pb1 * m + (1.0 - m) * f

        x0 = jnp.where(has_nw, jnp.where(hot_nw, d_nw[0], d_def[0]), d_def[0])
        x1 = jnp.where(has_nw, jnp.where(hot_nw, d_nw[1], d_def[1]), d_def[1])
        fills = [
            jnp.where(has_w, jnp.where(hot_w, d_w[0], x0), x0),
            jnp.where(has_w, jnp.where(hot_w, d_w[1], x1), x1),
            jnp.where(ht1_known,
                      jnp.where(ht1_hot, d_h1tt[0], d_h1tt_off[0]), d_h1tt[0]),
            jnp.where(ht1_known,
                      jnp.where(ht1_hot, d_h1tt[1], d_h1tt_off[1]), d_h1tt[1]),
            jnp.where(ht2_known,
                      jnp.where(ht2_hot, d_h2tt[0], d_h2tt_off[0]), d_h2tt[0]),
            jnp.where(ht2_known,
                      jnp.where(ht2_hot, d_h2tt[1], d_h2tt_off[1]), d_h2tt[1]),
        ]
        for j in range(6):
            m = ld(psmv, j)
            t = ld(psv, j) * m + (1.0 - m) * fills[j]
            psv[j, pl.ds(off, 16)] = t * m + (1.0 - m) * d_ps[j]
        return carry

    lax.fori_loop(0, G, group, 0)

    cps = [
        pltpu.async_copy(pbv, pb_out.at[:, pl.ds(base, RPW)], sem),
        pltpu.async_copy(psv, ps_out.at[:, pl.ds(base, RPW)], sem),
    ]
    for c in cps:
        c.wait()


def _tc_body(pb_ref, ps_ref, pbm_ref, psm_ref, prm_ref, pb_out_ref, ps_out_ref):
    pb = pb_ref[...]
    ps = ps_ref[...]
    pbm = pbm_ref[...]
    psm = psm_ref[...]
    prm = prm_ref[...]

    d_pb = prm[0:9, :]
    d_def, d_nw, d_w = prm[9:11, :], prm[11:13, :], prm[13:15, :]
    d_h1tt, d_h1tt_off = prm[15:17, :], prm[17:19, :]
    d_h1c, d_h1c_on, d_h1c_off = prm[19:22, :], prm[22:25, :], prm[25:28, :]
    d_h2tt, d_h2tt_off = prm[28:30, :], prm[30:32, :]
    d_h2c, d_h2c_on, d_h2c_off = prm[32:34, :], prm[34:36, :], prm[36:38, :]
    d_ps = prm[38:44, :]

    pb1 = pb * pbm + (1.0 - pbm) * d_pb

    cond_nw = (pbm[0:1, :] > 0.5) & (pb1[0:1, :] > 0.5)
    cond_w = (pbm[1:2, :] > 0.5) & (pb1[1:2, :] > 0.5)
    ht1_known = pbm[2:3, :] > 0.5
    ht1_hot = pb1[2:3, :] > 0.5
    ht1_on = ht1_known & ht1_hot
    ht1_off = ht1_known & (~ht1_hot)
    ht2_known = pbm[6:7, :] > 0.5
    ht2_hot = pb1[6:7, :] > 0.5
    ht2_on = ht2_known & ht2_hot
    ht2_off = ht2_known & (~ht2_hot)

    def_fill = jnp.where(cond_w, d_w, jnp.where(cond_nw, d_nw, d_def))
    ht1_tt = jnp.where(ht1_off, d_h1tt_off, d_h1tt)
    ht2_tt = jnp.where(ht2_off, d_h2tt_off, d_h2tt)
    ht1_cool = jnp.where(ht1_off, d_h1c_off,
                         jnp.where(ht1_on, d_h1c_on, d_h1c))
    ht2_cool = jnp.where(ht2_off, d_h2c_off,
                         jnp.where(ht2_on, d_h2c_on, d_h2c))

    pb_out_ref[0:3, :] = pb1[0:3, :]
    m36 = pbm[3:6, :]
    pb_out_ref[3:6, :] = pb1[3:6, :] * m36 + (1.0 - m36) * ht1_cool
    pb_out_ref[6:7, :] = pb1[6:7, :]
    m79 = pbm[7:9, :]
    pb_out_ref[7:9, :] = pb1[7:9, :] * m79 + (1.0 - m79) * ht2_cool

    fill_ps = jnp.concatenate([def_fill, ht1_tt, ht2_tt], axis=0)
    t = ps * psm + (1.0 - psm) * fill_ps
    ps_out_ref[...] = t * psm + (1.0 - psm) * d_ps


def kernel(proc_bool, proc_scalar, proc_bool_mask, proc_scalar_mask,
           p_pb_def, p_def_def, p_def_nw, p_def_w,
           p_ht1_tt_def, p_ht1_tt_off,
           p_ht1_cool_def, p_ht1_cool_on, p_ht1_cool_off,
           p_ht2_tt_def, p_ht2_tt_off,
           p_ht2_cool_def, p_ht2_cool_on, p_ht2_cool_off, p_ps_def):
    prm = jnp.concatenate(
        [p_pb_def, p_def_def, p_def_nw, p_def_w,
         p_ht1_tt_def, p_ht1_tt_off,
         p_ht1_cool_def, p_ht1_cool_on, p_ht1_cool_off,
         p_ht2_tt_def, p_ht2_tt_off,
         p_ht2_cool_def, p_ht2_cool_on, p_ht2_cool_off, p_ps_def])
    tbl = jnp.broadcast_to(prm[:, None], (44, 16))

    pbT, psT = proc_bool.T, proc_scalar.T
    pbmT, psmT = proc_bool_mask.T, proc_scalar_mask.T

    mesh = plsc.VectorSubcoreMesh(core_axis_name="c", subcore_axis_name="s")
    f32 = jnp.float32
    sck = pl.kernel(
        _sc_body,
        mesh=mesh,
        out_type=[jax.ShapeDtypeStruct((9, S), f32),
                  jax.ShapeDtypeStruct((6, S), f32)],
        scratch_types=[
            pltpu.VMEM((9, RPW), f32),
            pltpu.VMEM((6, RPW), f32),
            pltpu.VMEM((9, RPW), f32),
            pltpu.VMEM((6, RPW), f32),
            pltpu.VMEM((44, 16), f32),
            pltpu.SemaphoreType.DMA,
        ],
    )
    sc_pb, sc_ps = sck(pbT, psT, pbmT, psmT, tbl)

    grid = ((B - S) // CB,)
    off = S // CB
    col_spec9 = pl.BlockSpec((9, CB), lambda i: (0, i + off))
    col_spec6 = pl.BlockSpec((6, CB), lambda i: (0, i + off))
    out_spec9 = pl.BlockSpec((9, CB), lambda i: (0, i))
    out_spec6 = pl.BlockSpec((6, CB), lambda i: (0, i))
    prm_spec = pl.BlockSpec((44, 1), lambda i: (0, 0))

    tc_pb, tc_ps = pl.pallas_call(
        _tc_body,
        grid=grid,
        in_specs=[col_spec9, col_spec6, col_spec9, col_spec6, prm_spec],
        out_specs=[out_spec9, out_spec6],
        out_shape=[jax.ShapeDtypeStruct((9, B - S), f32),
                   jax.ShapeDtypeStruct((6, B - S), f32)],
        compiler_params=pltpu.CompilerParams(
            dimension_semantics=("parallel",)),
    )(pbT, psT, pbmT, psmT, prm[:, None])

    pb_out = jnp.concatenate([sc_pb, tc_pb], axis=1)
    ps_out = jnp.concatenate([sc_ps, tc_ps], axis=1)
    return (pb_out.T, ps_out.T)


# speedup vs baseline: 2.3130x; 2.3130x over previous
import jax
import jax.numpy as jnp
from jax.experimental import pallas as pl
from jax.experimental.pallas import tpu as pltpu

B = 16384
CB = 2048
NCH = B // CB


def _compute(pb, ps, pbm, psm, prm):
    d_pb = prm[0:9, :]
    d_def, d_nw, d_w = prm[9:11, :], prm[11:13, :], prm[13:15, :]
    d_h1tt, d_h1tt_off = prm[15:17, :], prm[17:19, :]
    d_h1c, d_h1c_on, d_h1c_off = prm[19:22, :], prm[22:25, :], prm[25:28, :]
    d_h2tt, d_h2tt_off = prm[28:30, :], prm[30:32, :]
    d_h2c, d_h2c_on, d_h2c_off = prm[32:34, :], prm[34:36, :], prm[36:38, :]
    d_ps = prm[38:44, :]

    pb1 = pb * pbm + (1.0 - pbm) * d_pb

    cond_nw = (pbm[0:1, :] > 0.5) & (pb1[0:1, :] > 0.5)
    cond_w = (pbm[1:2, :] > 0.5) & (pb1[1:2, :] > 0.5)
    ht1_known = pbm[2:3, :] > 0.5
    ht1_hot = pb1[2:3, :] > 0.5
    ht1_on = ht1_known & ht1_hot
    ht1_off = ht1_known & (~ht1_hot)
    ht2_known = pbm[6:7, :] > 0.5
    ht2_hot = pb1[6:7, :] > 0.5
    ht2_on = ht2_known & ht2_hot
    ht2_off = ht2_known & (~ht2_hot)

    def_fill = jnp.where(cond_w, d_w, jnp.where(cond_nw, d_nw, d_def))
    ht1_tt = jnp.where(ht1_off, d_h1tt_off, d_h1tt)
    ht2_tt = jnp.where(ht2_off, d_h2tt_off, d_h2tt)
    ht1_cool = jnp.where(ht1_off, d_h1c_off,
                         jnp.where(ht1_on, d_h1c_on, d_h1c))
    ht2_cool = jnp.where(ht2_off, d_h2c_off,
                         jnp.where(ht2_on, d_h2c_on, d_h2c))

    m36 = pbm[3:6, :]
    m79 = pbm[7:9, :]
    pb_mid = pb1[3:6, :] * m36 + (1.0 - m36) * ht1_cool
    pb_hi = pb1[7:9, :] * m79 + (1.0 - m79) * ht2_cool

    fill_ps = jnp.concatenate([def_fill, ht1_tt, ht2_tt], axis=0)
    t = ps * psm + (1.0 - psm) * fill_ps
    ps_out = t * psm + (1.0 - psm) * d_ps
    return pb1, pb_mid, pb_hi, ps_out


def _body(pb_hbm, ps_hbm, pbm_hbm, psm_hbm, prm_ref, pb_out_hbm, ps_out_hbm,
          pbb, psb, pbmb, psmb, pob, psob, sin, sout):
    prm = prm_ref[...]

    def issue_in(k, d):
        sl = pl.ds(k * CB, CB)
        cps = [
            pltpu.make_async_copy(pb_hbm.at[:, sl], pbb.at[d], sin.at[d, 0]),
            pltpu.make_async_copy(ps_hbm.at[:, sl], psb.at[d], sin.at[d, 1]),
            pltpu.make_async_copy(pbm_hbm.at[:, sl], pbmb.at[d], sin.at[d, 2]),
            pltpu.make_async_copy(psm_hbm.at[:, sl], psmb.at[d], sin.at[d, 3]),
        ]
        for c in cps:
            c.start()
        return cps

    def issue_out(k, d):
        sl = pl.ds(k * CB, CB)
        cps = [
            pltpu.make_async_copy(pob.at[d], pb_out_hbm.at[:, sl], sout.at[d, 0]),
            pltpu.make_async_copy(psob.at[d], ps_out_hbm.at[:, sl], sout.at[d, 1]),
        ]
        for c in cps:
            c.start()
        return cps

    hin = {0: issue_in(0, 0)}
    hout = [None, None]
    for k in range(NCH):
        d = k % 2
        for c in hin.pop(k):
            c.wait()
        if k + 1 < NCH:
            hin[k + 1] = issue_in(k + 1, (k + 1) % 2)
        if hout[d] is not None:
            for c in hout[d]:
                c.wait()
        pb1, pb_mid, pb_hi, ps_out = _compute(
            pbb[d], psb[d], pbmb[d], psmb[d], prm)
        pob[d, 0:3, :] = pb1[0:3, :]
        pob[d, 3:6, :] = pb_mid
        pob[d, 6:7, :] = pb1[6:7, :]
        pob[d, 7:9, :] = pb_hi
        psob[d] = ps_out
        hout[d] = issue_out(k, d)
    for d in (0, 1):
        if hout[d] is not None:
            for c in hout[d]:
                c.wait()


def kernel(proc_bool, proc_scalar, proc_bool_mask, proc_scalar_mask,
           p_pb_def, p_def_def, p_def_nw, p_def_w,
           p_ht1_tt_def, p_ht1_tt_off,
           p_ht1_cool_def, p_ht1_cool_on, p_ht1_cool_off,
           p_ht2_tt_def, p_ht2_tt_off,
           p_ht2_cool_def, p_ht2_cool_on, p_ht2_cool_off, p_ps_def):
    prm = jnp.concatenate(
        [p_pb_def, p_def_def, p_def_nw, p_def_w,
         p_ht1_tt_def, p_ht1_tt_off,
         p_ht1_cool_def, p_ht1_cool_on, p_ht1_cool_off,
         p_ht2_tt_def, p_ht2_tt_off,
         p_ht2_cool_def, p_ht2_cool_on, p_ht2_cool_off, p_ps_def])[:, None]

    f32 = jnp.float32
    any_spec = pl.BlockSpec(memory_space=pl.ANY)
    pb_out, ps_out = pl.pallas_call(
        _body,
        in_specs=[any_spec, any_spec, any_spec, any_spec,
                  pl.BlockSpec(memory_space=pltpu.VMEM)],
        out_specs=[any_spec, any_spec],
        out_shape=[jax.ShapeDtypeStruct((9, B), f32),
                   jax.ShapeDtypeStruct((6, B), f32)],
        scratch_shapes=[
            pltpu.VMEM((2, 9, CB), f32),
            pltpu.VMEM((2, 6, CB), f32),
            pltpu.VMEM((2, 9, CB), f32),
            pltpu.VMEM((2, 6, CB), f32),
            pltpu.VMEM((2, 9, CB), f32),
            pltpu.VMEM((2, 6, CB), f32),
            pltpu.SemaphoreType.DMA((2, 4)),
            pltpu.SemaphoreType.DMA((2, 2)),
        ],
    )(proc_bool.T, proc_scalar.T, proc_bool_mask.T, proc_scalar_mask.T, prm)
    return (pb_out.T, ps_out.T)


# speedup vs baseline: 4.1109x; 1.7774x over previous
import jax
import jax.numpy as jnp
from jax.experimental import pallas as pl
from jax.experimental.pallas import tpu as pltpu

B = 16384
CB = 8192


def _body(pb_ref, ps_ref, pbm_ref, psm_ref, prm_ref, pb_out_ref, ps_out_ref):
    pb = pb_ref[...]
    ps = ps_ref[...]
    pbm = pbm_ref[...]
    psm = psm_ref[...]
    prm = prm_ref[...]

    d_pb = prm[0:9, :]
    d_def, d_nw, d_w = prm[9:11, :], prm[11:13, :], prm[13:15, :]
    d_h1tt, d_h1tt_off = prm[15:17, :], prm[17:19, :]
    d_h1c, d_h1c_on, d_h1c_off = prm[19:22, :], prm[22:25, :], prm[25:28, :]
    d_h2tt, d_h2tt_off = prm[28:30, :], prm[30:32, :]
    d_h2c, d_h2c_on, d_h2c_off = prm[32:34, :], prm[34:36, :], prm[36:38, :]
    d_ps = prm[38:44, :]

    pb1 = pb * pbm + (1.0 - pbm) * d_pb

    cond_nw = (pbm[0:1, :] > 0.5) & (pb1[0:1, :] > 0.5)
    cond_w = (pbm[1:2, :] > 0.5) & (pb1[1:2, :] > 0.5)
    ht1_known = pbm[2:3, :] > 0.5
    ht1_hot = pb1[2:3, :] > 0.5
    ht1_on = ht1_known & ht1_hot
    ht1_off = ht1_known & (~ht1_hot)
    ht2_known = pbm[6:7, :] > 0.5
    ht2_hot = pb1[6:7, :] > 0.5
    ht2_on = ht2_known & ht2_hot
    ht2_off = ht2_known & (~ht2_hot)

    def_fill = jnp.where(cond_w, d_w, jnp.where(cond_nw, d_nw, d_def))
    ht1_tt = jnp.where(ht1_off, d_h1tt_off, d_h1tt)
    ht2_tt = jnp.where(ht2_off, d_h2tt_off, d_h2tt)
    ht1_cool = jnp.where(ht1_off, d_h1c_off,
                         jnp.where(ht1_on, d_h1c_on, d_h1c))
    ht2_cool = jnp.where(ht2_off, d_h2c_off,
                         jnp.where(ht2_on, d_h2c_on, d_h2c))

    pb_out_ref[0:3, :] = pb1[0:3, :]
    m36 = pbm[3:6, :]
    pb_out_ref[3:6, :] = pb1[3:6, :] * m36 + (1.0 - m36) * ht1_cool
    pb_out_ref[6:7, :] = pb1[6:7, :]
    m79 = pbm[7:9, :]
    pb_out_ref[7:9, :] = pb1[7:9, :] * m79 + (1.0 - m79) * ht2_cool

    fill_ps = jnp.concatenate([def_fill, ht1_tt, ht2_tt], axis=0)
    t = ps * psm + (1.0 - psm) * fill_ps
    ps_out_ref[...] = t * psm + (1.0 - psm) * d_ps


def kernel(proc_bool, proc_scalar, proc_bool_mask, proc_scalar_mask,
           p_pb_def, p_def_def, p_def_nw, p_def_w,
           p_ht1_tt_def, p_ht1_tt_off,
           p_ht1_cool_def, p_ht1_cool_on, p_ht1_cool_off,
           p_ht2_tt_def, p_ht2_tt_off,
           p_ht2_cool_def, p_ht2_cool_on, p_ht2_cool_off, p_ps_def):
    prm = jnp.concatenate(
        [p_pb_def, p_def_def, p_def_nw, p_def_w,
         p_ht1_tt_def, p_ht1_tt_off,
         p_ht1_cool_def, p_ht1_cool_on, p_ht1_cool_off,
         p_ht2_tt_def, p_ht2_tt_off,
         p_ht2_cool_def, p_ht2_cool_on, p_ht2_cool_off, p_ps_def])[:, None]

    grid = (B // CB,)
    col_spec9 = pl.BlockSpec((9, CB), lambda i: (0, i))
    col_spec6 = pl.BlockSpec((6, CB), lambda i: (0, i))
    prm_spec = pl.BlockSpec((44, 1), lambda i: (0, 0))

    pb_out, ps_out = pl.pallas_call(
        _body,
        grid=grid,
        in_specs=[col_spec9, col_spec6, col_spec9, col_spec6, prm_spec],
        out_specs=[col_spec9, col_spec6],
        out_shape=[jax.ShapeDtypeStruct((9, B), jnp.float32),
                   jax.ShapeDtypeStruct((6, B), jnp.float32)],
        compiler_params=pltpu.CompilerParams(
            dimension_semantics=("parallel",)),
    )(proc_bool.T, proc_scalar.T, proc_bool_mask.T, proc_scalar_mask.T, prm)
    return (pb_out.T, ps_out.T)
